# Initial kernel scaffold; baseline (speedup 1.0000x reference)
import jax
import jax.numpy as jnp
from jax.experimental import pallas as pl


def kernel(x, weight):
    raise NotImplementedError("write your pallas kernel here")


# speedup vs baseline: 1.4794x; 1.4794x over previous
import functools

import jax
import jax.numpy as jnp
from jax import lax
from jax.experimental import pallas as pl
from jax.experimental.pallas import tpu as pltpu
from jax.experimental.pallas import tpu_sc as plsc

_NC = 2
_NS = 16
_NW = _NC * _NS

_GL = 128
_K = 8


@functools.partial(jax.jit, static_argnums=(2, 3))
def _emb_lookup(x2d, weight, r_w, n_chunks):
    R = x2d.shape[0]
    D = weight.shape[1]
    mesh = plsc.VectorSubcoreMesh(core_axis_name="c", subcore_axis_name="s")

    @functools.partial(
        pl.kernel,
        out_type=jax.ShapeDtypeStruct((R, _GL, D), jnp.float32),
        mesh=mesh,
        compiler_params=pltpu.CompilerParams(use_tc_tiling_on_sc=False),
        scratch_types=[
            pltpu.VMEM((r_w, _GL), jnp.int32),
            pltpu.VMEM((_K, _GL, D), jnp.float32),
            pltpu.SemaphoreType.DMA,
        ],
    )
    def body(x_hbm, w_hbm, out_hbm, idx_v, rows_v, gsem):
        wid = lax.axis_index("s") * _NC + lax.axis_index("c")
        row0 = wid * r_w
        pltpu.sync_copy(x_hbm.at[pl.ds(row0, r_w)], idx_v)

        @pl.loop(0, n_chunks)
        def chunk(g):
            base = g * _K
            copies = [
                pltpu.async_copy(w_hbm.at[idx_v.at[base + j]], rows_v.at[j], gsem)
                for j in range(_K)
            ]
            for c in copies:
                c.wait()
            pltpu.sync_copy(rows_v, out_hbm.at[pl.ds(row0 + base, _K)])

    return body(x2d, weight)


def kernel(x, weight):
    B, L = x.shape
    V, D = weight.shape
    N = B * L
    assert N % (_NW * _GL * _K) == 0
    r_w = N // (_NW * _GL)
    n_chunks = r_w // _K
    x2d = x.astype(jnp.int32).reshape(N // _GL, _GL)
    out = _emb_lookup(x2d, weight, r_w, n_chunks)
    return out.reshape(B, L, D)


# speedup vs baseline: 1.5011x; 1.0147x over previous
import functools

import jax
import jax.numpy as jnp
from jax import lax
from jax.experimental import pallas as pl
from jax.experimental.pallas import tpu as pltpu
from jax.experimental.pallas import tpu_sc as plsc

_NC = 2
_NS = 16
_NW = _NC * _NS

_GL = 128
_K = 10


@functools.partial(jax.jit, static_argnums=(2, 3))
def _emb_lookup(x2d, weight, r_w, n_chunks):
    R = x2d.shape[0]
    D = weight.shape[1]
    mesh = plsc.VectorSubcoreMesh(core_axis_name="c", subcore_axis_name="s")

    @functools.partial(
        pl.kernel,
        out_type=jax.ShapeDtypeStruct((R, _GL, D), jnp.float32),
        mesh=mesh,
        compiler_params=pltpu.CompilerParams(use_tc_tiling_on_sc=False),
        scratch_types=[
            pltpu.VMEM((r_w, _GL), jnp.int32),
            pltpu.VMEM((2, _K, _GL, D), jnp.float32),
            pltpu.SemaphoreType.DMA,
            pltpu.SemaphoreType.DMA,
        ],
    )
    def body(x_hbm, w_hbm, out_hbm, idx_v, rows_v, sem0, sem1):
        sems = [sem0, sem1]
        wid = lax.axis_index("s") * _NC + lax.axis_index("c")
        row0 = wid * r_w
        pltpu.sync_copy(x_hbm.at[pl.ds(row0, r_w)], idx_v)

        def fire(c, b):
            base = c * _K
            for j in range(_K):
                pltpu.async_copy(
                    w_hbm.at[idx_v.at[base + j]], rows_v.at[b, j], sems[b])

        def drain_write(c, b):
            for j in range(_K):
                pltpu.make_async_copy(
                    w_hbm.at[idx_v.at[j]], rows_v.at[b, j], sems[b]).wait()
            pltpu.sync_copy(rows_v.at[b], out_hbm.at[pl.ds(row0 + c * _K, _K)])

        fire(0, 0)
        fire(1, 1)

        @pl.loop(0, n_chunks // 2 - 1)
        def lp(i):
            c = 2 * i
            drain_write(c, 0)
            fire(c + 2, 0)
            drain_write(c + 1, 1)
            fire(c + 3, 1)

        drain_write(n_chunks - 2, 0)
        drain_write(n_chunks - 1, 1)

    return body(x2d, weight)


def kernel(x, weight):
    B, L = x.shape
    V, D = weight.shape
    N = B * L
    assert N % (_NW * _GL * _K) == 0
    r_w = N // (_NW * _GL)
    n_chunks = r_w // _K
    assert n_chunks % 2 == 0
    x2d = x.astype(jnp.int32).reshape(N // _GL, _GL)
    out = _emb_lookup(x2d, weight, r_w, n_chunks)
    return out.reshape(B, L, D)
